# baseline (speedup 1.0000x reference)
import jax
import jax.numpy as jnp
from jax import lax
from jax.experimental import pallas as pl
from jax.experimental.pallas import tpu as pltpu
from jax.experimental.pallas import tpu_sc as plsc

_NC = 2
_NS = 16
_NW = _NC * _NS

_CHUNK = 128
_RING = 4

_TCOLS = 512


def _transpose_pad_body(wt_ref, out_ref):
    xt = jnp.swapaxes(wt_ref[...], 0, 1)
    out_ref[...] = jnp.concatenate(
        [xt, jnp.zeros((_TCOLS, 64), jnp.float32)], axis=1
    )


def _transpose_pad(wt):
    D, V = wt.shape
    grid = (V + _TCOLS - 1) // _TCOLS
    return pl.pallas_call(
        _transpose_pad_body,
        grid=(grid,),
        in_specs=[pl.BlockSpec((D, _TCOLS), lambda j: (0, j))],
        out_specs=pl.BlockSpec((_TCOLS, 2 * D), lambda j: (j, 0)),
        out_shape=jax.ShapeDtypeStruct((V, 2 * D), jnp.float32),
    )(wt)


def _body(idx_hbm, table_hbm, out_hbm, idx_v, rows_v, gsem, osem):
    cpw = idx_hbm.shape[1]
    wid = lax.axis_index("s") * _NC + lax.axis_index("c")
    pltpu.sync_copy(idx_hbm.at[wid], idx_v)
    out_base = wid * cpw

    def fire_gather(g, b):
        pltpu.async_copy(
            table_hbm.at[idx_v.at[g]], rows_v.at[b], gsem.at[b]
        )

    def valid(b):
        return rows_v.at[b, :, pl.ds(0, 64)]

    def out_slice(g):
        base = pl.multiple_of((out_base + g) * _CHUNK, _CHUNK)
        return out_hbm.at[pl.ds(base, _CHUNK), :]

    for b in range(_RING):
        fire_gather(b, b)

    @pl.loop(0, cpw - _RING, step=_RING)
    def _steady(i):
        for b in range(_RING):
            g = i + b
            pltpu.make_async_copy(
                table_hbm.at[pl.ds(0, _CHUNK)], rows_v.at[b], gsem.at[b]
            ).wait()
            pltpu.async_copy(valid(b), out_slice(g), osem.at[b])
            pltpu.make_async_copy(
                valid(b), out_hbm.at[pl.ds(0, _CHUNK), :], osem.at[b]
            ).wait()
            fire_gather(g + _RING, b)

    for b in range(_RING):
        g = cpw - _RING + b
        pltpu.make_async_copy(
            table_hbm.at[pl.ds(0, _CHUNK)], rows_v.at[b], gsem.at[b]
        ).wait()
        pltpu.sync_copy(valid(b), out_slice(g))


def kernel(token_ids, W):
    S, T = token_ids.shape
    D = W.shape[1]
    B = S * T
    rows_per_w = B // _NW
    idx = token_ids.astype(jnp.int32).reshape(_NW, rows_per_w // _CHUNK,
                                              _CHUNK)

    w_padded = _transpose_pad(W.T)

    mesh = plsc.VectorSubcoreMesh(core_axis_name="c", subcore_axis_name="s")
    run = pl.kernel(
        _body,
        out_type=jax.ShapeDtypeStruct((B, D), jnp.float32),
        mesh=mesh,
        scratch_types=[
            pltpu.VMEM((rows_per_w // _CHUNK, _CHUNK), jnp.int32),
            pltpu.VMEM((_RING, _CHUNK, 2 * D), jnp.float32),
            pltpu.SemaphoreType.DMA((_RING,)),
            pltpu.SemaphoreType.DMA((_RING,)),
        ],
        compiler_params=pltpu.CompilerParams(use_tc_tiling_on_sc=False),
    )
    out = run(idx, w_padded)
    return out.reshape(S, T, D)


# speedup vs baseline: 1.2556x; 1.2556x over previous
import jax
import jax.numpy as jnp
from jax import lax
from jax.experimental import pallas as pl
from jax.experimental.pallas import tpu as pltpu
from jax.experimental.pallas import tpu_sc as plsc

_NC = 2
_NS = 16
_NW = _NC * _NS

_CHUNK = 128
_RING = 4

_TROWS = 1600


def _pad_copy_body(w_ref, out_ref):
    out_ref[...] = jnp.concatenate(
        [w_ref[...], jnp.zeros((_TROWS, 64), jnp.float32)], axis=1
    )


def _pad_copy(w):
    V, D = w.shape
    grid = V // _TROWS
    return pl.pallas_call(
        _pad_copy_body,
        grid=(grid,),
        in_specs=[pl.BlockSpec((_TROWS, D), lambda j: (j, 0))],
        out_specs=pl.BlockSpec((_TROWS, 2 * D), lambda j: (j, 0)),
        out_shape=jax.ShapeDtypeStruct((V, 2 * D), jnp.float32),
    )(w)


def _body(idx_hbm, table_hbm, out_hbm, idx_v, rows_v, gsem, osem):
    cpw = idx_hbm.shape[1]
    wid = lax.axis_index("s") * _NC + lax.axis_index("c")
    pltpu.sync_copy(idx_hbm.at[wid], idx_v)
    out_base = wid * cpw

    def fire_gather(g, b):
        pltpu.async_copy(
            table_hbm.at[idx_v.at[g]], rows_v.at[b], gsem.at[b]
        )

    def valid(b):
        return rows_v.at[b, :, pl.ds(0, 64)]

    def out_slice(g):
        base = pl.multiple_of((out_base + g) * _CHUNK, _CHUNK)
        return out_hbm.at[pl.ds(base, _CHUNK), :]

    for b in range(_RING):
        fire_gather(b, b)

    @pl.loop(0, cpw - _RING, step=_RING)
    def _steady(i):
        for b in range(_RING):
            g = i + b
            pltpu.make_async_copy(
                table_hbm.at[pl.ds(0, _CHUNK)], rows_v.at[b], gsem.at[b]
            ).wait()
            pltpu.async_copy(valid(b), out_slice(g), osem.at[b])
            pltpu.make_async_copy(
                valid(b), out_hbm.at[pl.ds(0, _CHUNK), :], osem.at[b]
            ).wait()
            fire_gather(g + _RING, b)

    for b in range(_RING):
        g = cpw - _RING + b
        pltpu.make_async_copy(
            table_hbm.at[pl.ds(0, _CHUNK)], rows_v.at[b], gsem.at[b]
        ).wait()
        pltpu.sync_copy(valid(b), out_slice(g))


def kernel(token_ids, W):
    S, T = token_ids.shape
    D = W.shape[1]
    B = S * T
    rows_per_w = B // _NW
    idx = token_ids.astype(jnp.int32).reshape(_NW, rows_per_w // _CHUNK,
                                              _CHUNK)

    w_padded = _pad_copy(W)

    mesh = plsc.VectorSubcoreMesh(core_axis_name="c", subcore_axis_name="s")
    run = pl.kernel(
        _body,
        out_type=jax.ShapeDtypeStruct((B, D), jnp.float32),
        mesh=mesh,
        scratch_types=[
            pltpu.VMEM((rows_per_w // _CHUNK, _CHUNK), jnp.int32),
            pltpu.VMEM((_RING, _CHUNK, 2 * D), jnp.float32),
            pltpu.SemaphoreType.DMA((_RING,)),
            pltpu.SemaphoreType.DMA((_RING,)),
        ],
        compiler_params=pltpu.CompilerParams(use_tc_tiling_on_sc=False),
    )
    out = run(idx, w_padded)
    return out.reshape(S, T, D)


# speedup vs baseline: 1.4036x; 1.1179x over previous
import jax
import jax.numpy as jnp
from jax import lax
from jax.experimental import pallas as pl
from jax.experimental.pallas import tpu as pltpu
from jax.experimental.pallas import tpu_sc as plsc

_NC = 2
_NS = 16
_NW = _NC * _NS

_CHUNK = 128
_RING = 4

_TROWS = 200


def _pad_copy_body(w_ref, out_ref):
    out_ref[...] = jnp.concatenate(
        [w_ref[...], jnp.zeros((_TROWS, 8, 64), jnp.float32)], axis=2
    )


def _pad_copy(w):
    G = w.shape[0]
    grid = G // _TROWS
    return pl.pallas_call(
        _pad_copy_body,
        grid=(grid,),
        in_specs=[pl.BlockSpec((_TROWS, 8, 64), lambda j: (j, 0, 0))],
        out_specs=pl.BlockSpec((_TROWS, 8, 128), lambda j: (j, 0, 0)),
        out_shape=jax.ShapeDtypeStruct((G, 8, 128), jnp.float32),
    )(w)


def _body(idx_hbm, table_hbm, out_hbm, idx_v, rows_v, gsem, osem):
    cpw = idx_hbm.shape[1]
    wid = lax.axis_index("s") * _NC + lax.axis_index("c")
    pltpu.sync_copy(idx_hbm.at[wid], idx_v)
    out_base = wid * cpw

    def fire_gather(g, b):
        pltpu.async_copy(
            table_hbm.at[idx_v.at[g]], rows_v.at[b], gsem.at[b]
        )

    def valid(b):
        return rows_v.at[b, :, pl.ds(0, 64)]

    def out_slice(g):
        base = pl.multiple_of((out_base + g) * _CHUNK, _CHUNK)
        return out_hbm.at[pl.ds(base, _CHUNK), :]

    for b in range(_RING):
        fire_gather(b, b)

    @pl.loop(0, cpw - _RING, step=_RING)
    def _steady(i):
        for b in range(_RING):
            g = i + b
            pltpu.make_async_copy(
                table_hbm.at[pl.ds(0, _CHUNK)], rows_v.at[b], gsem.at[b]
            ).wait()
            pltpu.async_copy(valid(b), out_slice(g), osem.at[b])
            pltpu.make_async_copy(
                valid(b), out_hbm.at[pl.ds(0, _CHUNK), :], osem.at[b]
            ).wait()
            fire_gather(g + _RING, b)

    for b in range(_RING):
        g = cpw - _RING + b
        pltpu.make_async_copy(
            table_hbm.at[pl.ds(0, _CHUNK)], rows_v.at[b], gsem.at[b]
        ).wait()
        pltpu.sync_copy(valid(b), out_slice(g))


def kernel(token_ids, W):
    S, T = token_ids.shape
    D = W.shape[1]
    B = S * T
    rows_per_w = B // _NW
    idx = token_ids.astype(jnp.int32).reshape(_NW, rows_per_w // _CHUNK,
                                              _CHUNK)

    w_grouped = W.reshape(W.shape[0] // 8, 8, D)
    w_padded = _pad_copy(w_grouped).reshape(W.shape[0], 2 * D)

    mesh = plsc.VectorSubcoreMesh(core_axis_name="c", subcore_axis_name="s")
    run = pl.kernel(
        _body,
        out_type=jax.ShapeDtypeStruct((B, D), jnp.float32),
        mesh=mesh,
        scratch_types=[
            pltpu.VMEM((rows_per_w // _CHUNK, _CHUNK), jnp.int32),
            pltpu.VMEM((_RING, _CHUNK, 2 * D), jnp.float32),
            pltpu.SemaphoreType.DMA((_RING,)),
            pltpu.SemaphoreType.DMA((_RING,)),
        ],
        compiler_params=pltpu.CompilerParams(use_tc_tiling_on_sc=False),
    )
    out = run(idx, w_padded)
    return out.reshape(S, T, D)


# speedup vs baseline: 1.7237x; 1.2281x over previous
import jax
import jax.numpy as jnp
from jax import lax
from jax.experimental import pallas as pl
from jax.experimental.pallas import tpu as pltpu
from jax.experimental.pallas import tpu_sc as plsc

_NC = 2
_NS = 16
_NW = _NC * _NS

_CHUNK = 128
_RING = 4

_TROWS = 1000


def _pad_copy_body(w_ref, out_ref):
    out_ref[:, :, 0:64] = w_ref[...]


def _pad_copy(w):
    G = w.shape[0]
    grid = G // _TROWS
    return pl.pallas_call(
        _pad_copy_body,
        grid=(grid,),
        in_specs=[pl.BlockSpec((_TROWS, 8, 64), lambda j: (j, 0, 0))],
        out_specs=pl.BlockSpec((_TROWS, 8, 128), lambda j: (j, 0, 0)),
        out_shape=jax.ShapeDtypeStruct((G, 8, 128), jnp.float32),
    )(w)


def _body(idx_hbm, table_hbm, out_hbm, idx_v, rows_v, gsem, osem):
    cpw = idx_hbm.shape[1]
    wid = lax.axis_index("s") * _NC + lax.axis_index("c")
    pltpu.sync_copy(idx_hbm.at[wid], idx_v)
    out_base = wid * cpw

    def fire_gather(g, b):
        pltpu.async_copy(
            table_hbm.at[idx_v.at[g]], rows_v.at[b], gsem.at[b]
        )

    def valid(b):
        return rows_v.at[b, :, pl.ds(0, 64)]

    def out_slice(g):
        base = pl.multiple_of((out_base + g) * _CHUNK, _CHUNK)
        return out_hbm.at[pl.ds(base, _CHUNK), :]

    for b in range(_RING):
        fire_gather(b, b)

    @pl.loop(0, cpw - _RING, step=_RING)
    def _steady(i):
        for b in range(_RING):
            g = i + b
            pltpu.make_async_copy(
                table_hbm.at[pl.ds(0, _CHUNK)], rows_v.at[b], gsem.at[b]
            ).wait()
            pltpu.async_copy(valid(b), out_slice(g), osem.at[b])
            pltpu.make_async_copy(
                valid(b), out_hbm.at[pl.ds(0, _CHUNK), :], osem.at[b]
            ).wait()
            fire_gather(g + _RING, b)

    for b in range(_RING):
        g = cpw - _RING + b
        pltpu.make_async_copy(
            table_hbm.at[pl.ds(0, _CHUNK)], rows_v.at[b], gsem.at[b]
        ).wait()
        pltpu.sync_copy(valid(b), out_slice(g))


def kernel(token_ids, W):
    S, T = token_ids.shape
    D = W.shape[1]
    B = S * T
    rows_per_w = B // _NW
    idx = token_ids.astype(jnp.int32).reshape(_NW, rows_per_w // _CHUNK,
                                              _CHUNK)

    w_grouped = W.reshape(W.shape[0] // 8, 8, D)
    w_padded = _pad_copy(w_grouped).reshape(W.shape[0], 2 * D)

    mesh = plsc.VectorSubcoreMesh(core_axis_name="c", subcore_axis_name="s")
    run = pl.kernel(
        _body,
        out_type=jax.ShapeDtypeStruct((B, D), jnp.float32),
        mesh=mesh,
        scratch_types=[
            pltpu.VMEM((rows_per_w // _CHUNK, _CHUNK), jnp.int32),
            pltpu.VMEM((_RING, _CHUNK, 2 * D), jnp.float32),
            pltpu.SemaphoreType.DMA((_RING,)),
            pltpu.SemaphoreType.DMA((_RING,)),
        ],
        compiler_params=pltpu.CompilerParams(use_tc_tiling_on_sc=False),
    )
    out = run(idx, w_padded)
    return out.reshape(S, T, D)


# speedup vs baseline: 1.7302x; 1.0037x over previous
import jax
import jax.numpy as jnp
from jax import lax
from jax.experimental import pallas as pl
from jax.experimental.pallas import tpu as pltpu
from jax.experimental.pallas import tpu_sc as plsc

_NC = 2
_NS = 16
_NW = _NC * _NS

_CHUNK = 128
_RING = 4

_TROWS = 2500


def _pad_copy_body(w_ref, out_ref):
    out_ref[:, :, 0:64] = w_ref[...]


def _pad_copy(w):
    G = w.shape[0]
    grid = G // _TROWS
    return pl.pallas_call(
        _pad_copy_body,
        grid=(grid,),
        in_specs=[pl.BlockSpec((_TROWS, 8, 64), lambda j: (j, 0, 0))],
        out_specs=pl.BlockSpec((_TROWS, 8, 128), lambda j: (j, 0, 0)),
        out_shape=jax.ShapeDtypeStruct((G, 8, 128), jnp.float32),
    )(w)


def _body(idx_hbm, table_hbm, out_hbm, idx_v, rows_v, gsem, osem):
    cpw = idx_hbm.shape[1]
    wid = lax.axis_index("s") * _NC + lax.axis_index("c")
    pltpu.sync_copy(idx_hbm.at[wid], idx_v)
    out_base = wid * cpw

    def fire_gather(g, b):
        pltpu.async_copy(
            table_hbm.at[idx_v.at[g]], rows_v.at[b], gsem.at[b]
        )

    def valid(b):
        return rows_v.at[b, :, pl.ds(0, 64)]

    def out_slice(g):
        base = pl.multiple_of((out_base + g) * _CHUNK, _CHUNK)
        return out_hbm.at[pl.ds(base, _CHUNK), :]

    for b in range(_RING):
        fire_gather(b, b)

    @pl.loop(0, cpw - _RING, step=_RING)
    def _steady(i):
        for b in range(_RING):
            g = i + b
            pltpu.make_async_copy(
                table_hbm.at[pl.ds(0, _CHUNK)], rows_v.at[b], gsem.at[b]
            ).wait()
            pltpu.async_copy(valid(b), out_slice(g), osem.at[b])
            pltpu.make_async_copy(
                valid(b), out_hbm.at[pl.ds(0, _CHUNK), :], osem.at[b]
            ).wait()
            fire_gather(g + _RING, b)

    for b in range(_RING):
        g = cpw - _RING + b
        pltpu.make_async_copy(
            table_hbm.at[pl.ds(0, _CHUNK)], rows_v.at[b], gsem.at[b]
        ).wait()
        pltpu.sync_copy(valid(b), out_slice(g))


def kernel(token_ids, W):
    S, T = token_ids.shape
    D = W.shape[1]
    B = S * T
    rows_per_w = B // _NW
    idx = token_ids.astype(jnp.int32).reshape(_NW, rows_per_w // _CHUNK,
                                              _CHUNK)

    w_grouped = W.reshape(W.shape[0] // 8, 8, D)
    w_padded = _pad_copy(w_grouped).reshape(W.shape[0], 2 * D)

    mesh = plsc.VectorSubcoreMesh(core_axis_name="c", subcore_axis_name="s")
    run = pl.kernel(
        _body,
        out_type=jax.ShapeDtypeStruct((B, D), jnp.float32),
        mesh=mesh,
        scratch_types=[
            pltpu.VMEM((rows_per_w // _CHUNK, _CHUNK), jnp.int32),
            pltpu.VMEM((_RING, _CHUNK, 2 * D), jnp.float32),
            pltpu.SemaphoreType.DMA((_RING,)),
            pltpu.SemaphoreType.DMA((_RING,)),
        ],
        compiler_params=pltpu.CompilerParams(use_tc_tiling_on_sc=False),
    )
    out = run(idx, w_padded)
    return out.reshape(S, T, D)
